# baseline (speedup 1.0000x reference)
import functools
import jax
import jax.numpy as jnp
from jax import lax
from jax.experimental import pallas as pl
from jax.experimental.pallas import tpu as pltpu
from jax.experimental.pallas import tpu_sc as plsc

BN = 512
BT = 512
BE = 512
NW = 32
CH_A = 128
G = 10


def _sc_mesh():
    return plsc.VectorSubcoreMesh(core_axis_name="c", subcore_axis_name="s")


def _make_gather_rows(H, M, dtype):
    rpw = M // NW
    nch = rpw // CH_A
    NBUF = 3

    @functools.partial(
        pl.kernel,
        out_type=jax.ShapeDtypeStruct((M, H), dtype),
        mesh=_sc_mesh(),
        scratch_types=(
            [pltpu.VMEM((nch, CH_A), jnp.int32)]
            + [pltpu.VMEM((CH_A,), jnp.int32) for _ in range(NBUF)]
            + [pltpu.VMEM((CH_A, H), dtype) for _ in range(NBUF)]
            + [pltpu.SemaphoreType.DMA for _ in range(NBUF)]
        ),
    )
    def gather_rows(table_hbm, idx_hbm, out_hbm, idx_v, *rest):
        idxb = rest[:NBUF]
        bufs = rest[NBUF:2 * NBUF]
        sems = rest[2 * NBUF:3 * NBUF]
        wid = lax.axis_index("s") * 2 + lax.axis_index("c")
        base = wid * rpw
        pltpu.sync_copy(idx_hbm.at[wid], idx_v)

        def fire(ck, b):
            for v in range(CH_A // 16):
                idxb[b][pl.ds(v * 16, 16)] = idx_v[ck, pl.ds(v * 16, 16)]
            pltpu.make_async_copy(
                table_hbm.at[idxb[b]], bufs[b], sems[b]).start()

        for b in range(NBUF):
            fire(b, b)
        for ck in range(nch):
            b = ck % NBUF
            pltpu.make_async_copy(
                table_hbm.at[idxb[b]], bufs[b], sems[b]).wait()
            pltpu.sync_copy(
                bufs[b], out_hbm.at[pl.ds(base + ck * CH_A, CH_A)])
            if ck + NBUF < nch:
                fire(ck + NBUF, b)

    return gather_rows


def _lstm_chunk_body(kv_ref, xt_ref, h_ref, c_ref, wih_ref, whh_ref, b_ref,
                     h_out, c_out, gates_ref):
    i = pl.program_id(0)
    hdim = h_ref.shape[1]
    h = h_ref[...]
    c = c_ref[...]
    for g in range(G):
        gates_ref[...] = (
            jnp.dot(h.astype(jnp.bfloat16), whh_ref[...],
                    preferred_element_type=jnp.float32)
            + b_ref[...]
        )

        @pl.when(i * BN < kv_ref[g])
        def _():
            gates_ref[...] += jnp.dot(
                xt_ref[g].astype(jnp.bfloat16), wih_ref[...],
                preferred_element_type=jnp.float32)

        gt = gates_ref[...]
        gi = jax.nn.sigmoid(gt[:, 0 * hdim:1 * hdim])
        gf = jax.nn.sigmoid(gt[:, 1 * hdim:2 * hdim])
        gg = jnp.tanh(gt[:, 2 * hdim:3 * hdim])
        go = jax.nn.sigmoid(gt[:, 3 * hdim:4 * hdim])
        c_new = gf * c + gi * gg
        h_new = go * jnp.tanh(c_new)
        live = kv_ref[G + g] > 0
        h = jnp.where(live, h_new, h)
        c = jnp.where(live, c_new, c)
    h_out[...] = h
    c_out[...] = c


def _make_lstm_chunk(NP, H):
    NB = NP // BN

    def xt_map(i, kv):
        last = jnp.maximum(pl.cdiv(kv[0], BN) - 1, 0)
        return (0, jnp.minimum(i, last), 0)

    grid_spec = pltpu.PrefetchScalarGridSpec(
        num_scalar_prefetch=1,
        grid=(NB,),
        in_specs=[
            pl.BlockSpec((G, BN, H), xt_map),
            pl.BlockSpec((BN, H), lambda i, kv: (i, 0)),
            pl.BlockSpec((BN, H), lambda i, kv: (i, 0)),
            pl.BlockSpec((H, 4 * H), lambda i, kv: (0, 0)),
            pl.BlockSpec((H, 4 * H), lambda i, kv: (0, 0)),
            pl.BlockSpec((1, 4 * H), lambda i, kv: (0, 0)),
        ],
        out_specs=[
            pl.BlockSpec((BN, H), lambda i, kv: (i, 0)),
            pl.BlockSpec((BN, H), lambda i, kv: (i, 0)),
        ],
        scratch_shapes=[pltpu.VMEM((BN, 4 * H), jnp.float32)],
    )
    return pl.pallas_call(
        _lstm_chunk_body,
        grid_spec=grid_spec,
        out_shape=[
            jax.ShapeDtypeStruct((NP, H), jnp.float32),
            jax.ShapeDtypeStruct((NP, H), jnp.float32),
        ],
        compiler_params=pltpu.CompilerParams(
            dimension_semantics=("arbitrary",)),
    )


def _tail_body(aggr_ref, h_ref, wl_ref, wr_ref, b_ref, o_ref, *, nvalid):
    i = pl.program_id(0)
    v = (jnp.dot(aggr_ref[...].astype(jnp.bfloat16), wl_ref[...],
                 preferred_element_type=jnp.float32)
         + jnp.dot(h_ref[...].astype(jnp.bfloat16), wr_ref[...],
                   preferred_element_type=jnp.float32)
         + b_ref[...])
    v = jnp.maximum(v, 0.0)
    rows = i * BT + lax.broadcasted_iota(jnp.int32, v.shape, 0)
    o_ref[...] = jnp.where(rows < nvalid, v, 0.0)


def _make_tail(NP, H, N):
    return pl.pallas_call(
        functools.partial(_tail_body, nvalid=N),
        grid=(NP // BT,),
        in_specs=[
            pl.BlockSpec((BT, H), lambda i: (i, 0)),
            pl.BlockSpec((BT, H), lambda i: (i, 0)),
            pl.BlockSpec((H, H), lambda i: (0, 0)),
            pl.BlockSpec((H, H), lambda i: (0, 0)),
            pl.BlockSpec((1, H), lambda i: (0, 0)),
        ],
        out_specs=pl.BlockSpec((BT, H), lambda i: (i, 0)),
        out_shape=jax.ShapeDtypeStruct((NP, H), jnp.float32),
        compiler_params=pltpu.CompilerParams(
            dimension_semantics=("arbitrary",)),
    )


def _mlp_body(hs_ref, hd_ref, ea_ref, w1s_ref, w1d_ref, w1e_ref, b1_ref,
              w2_ref, b2_ref, w3_ref, b3_ref, o_ref):
    z = (jnp.dot(hs_ref[...].astype(jnp.bfloat16), w1s_ref[...],
                 preferred_element_type=jnp.float32)
         + jnp.dot(hd_ref[...].astype(jnp.bfloat16), w1d_ref[...],
                   preferred_element_type=jnp.float32)
         + jnp.dot(ea_ref[...], w1e_ref[...],
                   preferred_element_type=jnp.float32)
         + b1_ref[...])
    z = jnp.maximum(z, 0.0).astype(jnp.bfloat16)
    z = jnp.maximum(
        jnp.dot(z, w2_ref[...], preferred_element_type=jnp.float32)
        + b2_ref[...], 0.0).astype(jnp.bfloat16)
    o_ref[...] = (jnp.dot(z, w3_ref[...], preferred_element_type=jnp.float32)
                  + b3_ref[...])


def _make_mlp(EP, H, ED, H2, OUT):
    return pl.pallas_call(
        _mlp_body,
        grid=(EP // BE,),
        in_specs=[
            pl.BlockSpec((BE, H), lambda i: (i, 0)),
            pl.BlockSpec((BE, H), lambda i: (EP // BE + i, 0)),
            pl.BlockSpec((BE, ED), lambda i: (i, 0)),
            pl.BlockSpec((H, H), lambda i: (0, 0)),
            pl.BlockSpec((H, H), lambda i: (0, 0)),
            pl.BlockSpec((ED, H), lambda i: (0, 0)),
            pl.BlockSpec((1, H), lambda i: (0, 0)),
            pl.BlockSpec((H, H2), lambda i: (0, 0)),
            pl.BlockSpec((1, H2), lambda i: (0, 0)),
            pl.BlockSpec((H2, OUT), lambda i: (0, 0)),
            pl.BlockSpec((1, OUT), lambda i: (0, 0)),
        ],
        out_specs=pl.BlockSpec((BE, OUT), lambda i: (i, 0)),
        out_shape=jax.ShapeDtypeStruct((EP, OUT), jnp.float32),
        compiler_params=pltpu.CompilerParams(
            dimension_semantics=("arbitrary",)),
    )


def kernel(x, edge_index, edge_attr, params):
    x = x.astype(jnp.float32)
    src = edge_index[0].astype(jnp.int32)
    dst = edge_index[1].astype(jnp.int32)
    N, D = x.shape
    E = src.shape[0]
    H = D
    NP = -(-N // 2560) * 2560
    EPAD = -(-E // (NW * CH_A * 2)) * (NW * CH_A * 2)
    EP = EPAD
    assert EP % BE == 0

    starts_all = jnp.searchsorted(
        dst, jnp.arange(N, dtype=jnp.int32)).astype(jnp.int32)
    counts = jnp.diff(jnp.concatenate(
        [starts_all, jnp.array([E], jnp.int32)]))
    T = counts.max().astype(jnp.int32)

    order = jnp.argsort(-counts).astype(jnp.int32)
    counts_s = jnp.concatenate(
        [counts[order], jnp.zeros((NP - N,), jnp.int32)])
    starts_s = jnp.concatenate(
        [starts_all[order], jnp.full((NP - N,), E - 1, jnp.int32)])
    counts_asc = counts_s[::-1]
    pos = jnp.argsort(order).astype(jnp.int32)
    pos_pad = jnp.concatenate([pos, jnp.full((NP - N,), N, jnp.int32)])
    x_s = jnp.concatenate([x[order], jnp.zeros((NP - N, D), jnp.float32)])
    x_unp = jnp.concatenate([x, jnp.zeros((NP - N, D), jnp.float32)])

    def pad_idx(ix):
        return jnp.concatenate(
            [ix, jnp.full((EPAD - E,), N, jnp.int32)]
        ).reshape(NW, EPAD // NW // CH_A, CH_A)

    idx_src = pad_idx(src)
    idx_dst = pad_idx(dst)

    gather_rows = _make_gather_rows(H, EPAD, jnp.float32)
    lstm_chunk = _make_lstm_chunk(NP, H)
    tail = _make_tail(NP, H, N)
    garange = jnp.arange(G, dtype=jnp.int32)

    def layer(h_in, h_unp, p):
        wihT = p['W_ih'].T.astype(jnp.bfloat16)
        whhT = p['W_hh'].T.astype(jnp.bfloat16)
        b = (p['b_ih'] + p['b_hh']).reshape(1, 4 * H)

        feats = gather_rows(h_unp, idx_src)

        def gather_chunk(t0):
            ts = t0 + garange
            gidx = jnp.minimum(starts_s[None, :] + ts[:, None], E - 1)
            valid = ts[:, None] < counts_s[None, :]
            return jnp.where(valid[..., None], feats[gidx], 0.0)

        def cond(carry):
            t0 = carry[0]
            return t0 < T

        def body(carry):
            t0, h, c, xt = carry
            xt_next = gather_chunk(t0 + G)
            ts = t0 + garange
            kvec = (NP - jnp.searchsorted(counts_asc, ts, side='right')
                    ).astype(jnp.int32)
            live = (ts < T).astype(jnp.int32)
            kv = jnp.concatenate([kvec, live])
            h, c = lstm_chunk(kv, xt, h, c, wihT, whhT, b)
            return t0 + G, h, c, xt_next

        z = jnp.zeros((NP, H), jnp.float32)
        _, hl, _, _ = lax.while_loop(
            cond, body, (jnp.int32(0), z, z, gather_chunk(jnp.int32(0))))
        return tail(hl, h_in, p['W_l'].T.astype(jnp.bfloat16),
                    p['W_r'].T.astype(jnp.bfloat16), p['b_l'].reshape(1, H))

    h = layer(x_s, x_unp, params['conv1'])
    h = layer(h, h[pos_pad], params['conv2'])
    h = layer(h, h[pos_pad], params['conv3'])
    h_unp = h[pos_pad]

    gather_rows2 = _make_gather_rows(H, 2 * EPAD, jnp.float32)
    idx_both = jnp.concatenate(
        [idx_src.reshape(-1), idx_dst.reshape(-1)]
    ).reshape(NW, 2 * EPAD // NW // CH_A, CH_A)
    hsd = gather_rows2(h_unp, idx_both)
    m = params['edge_mlp']
    H2 = m['W2'].shape[0]
    OUT = m['W3'].shape[0]
    ED = edge_attr.shape[1]
    w1 = m['W1'].T.astype(jnp.bfloat16)

    ea = jnp.concatenate(
        [edge_attr.astype(jnp.bfloat16),
         jnp.zeros((EP - E, ED), jnp.bfloat16)])

    mlp = _make_mlp(EP, H, ED, H2, OUT)
    out = mlp(hsd, hsd, ea, w1[:H], w1[H:2 * H], w1[2 * H:],
              m['b1'].reshape(1, H),
              m['W2'].T.astype(jnp.bfloat16), m['b2'].reshape(1, H2),
              m['W3'].T.astype(jnp.bfloat16), m['b3'].reshape(1, OUT))
    return out[:E]


# speedup vs baseline: 1.0475x; 1.0475x over previous
import functools
import jax
import jax.numpy as jnp
from jax import lax
from jax.experimental import pallas as pl
from jax.experimental.pallas import tpu as pltpu
from jax.experimental.pallas import tpu_sc as plsc

BN = 512
BT = 512
BE = 512
NW = 32
CH_A = 128
G = 8


def _sc_mesh():
    return plsc.VectorSubcoreMesh(core_axis_name="c", subcore_axis_name="s")


def _make_gather_rows(H, M, dtype):
    rpw = M // NW
    nch = rpw // CH_A
    NBUF = 3

    @functools.partial(
        pl.kernel,
        out_type=jax.ShapeDtypeStruct((M, H), dtype),
        mesh=_sc_mesh(),
        scratch_types=(
            [pltpu.VMEM((nch, CH_A), jnp.int32)]
            + [pltpu.VMEM((CH_A,), jnp.int32) for _ in range(NBUF)]
            + [pltpu.VMEM((CH_A, H), dtype) for _ in range(NBUF)]
            + [pltpu.SemaphoreType.DMA for _ in range(NBUF)]
        ),
    )
    def gather_rows(table_hbm, idx_hbm, out_hbm, idx_v, *rest):
        idxb = rest[:NBUF]
        bufs = rest[NBUF:2 * NBUF]
        sems = rest[2 * NBUF:3 * NBUF]
        wid = lax.axis_index("s") * 2 + lax.axis_index("c")
        base = wid * rpw
        pltpu.sync_copy(idx_hbm.at[wid], idx_v)

        def fire(ck, b):
            for v in range(CH_A // 16):
                idxb[b][pl.ds(v * 16, 16)] = idx_v[ck, pl.ds(v * 16, 16)]
            pltpu.make_async_copy(
                table_hbm.at[idxb[b]], bufs[b], sems[b]).start()

        for b in range(NBUF):
            fire(b, b)
        for ck in range(nch):
            b = ck % NBUF
            pltpu.make_async_copy(
                table_hbm.at[idxb[b]], bufs[b], sems[b]).wait()
            pltpu.sync_copy(
                bufs[b], out_hbm.at[pl.ds(base + ck * CH_A, CH_A)])
            if ck + NBUF < nch:
                fire(ck + NBUF, b)

    return gather_rows


def _lstm_chunk_body(kv_ref, xt_ref, h_ref, c_ref, wih_ref, whh_ref, b_ref,
                     h_out, c_out, gates_ref):
    i = pl.program_id(0)
    hdim = h_ref.shape[1]
    h = h_ref[...]
    c = c_ref[...]
    for g in range(G):
        gates_ref[...] = (
            jnp.dot(h.astype(jnp.bfloat16), whh_ref[...],
                    preferred_element_type=jnp.float32)
            + b_ref[...]
        )

        @pl.when(i * BN < kv_ref[g])
        def _():
            gates_ref[...] += jnp.dot(
                xt_ref[g].astype(jnp.bfloat16), wih_ref[...],
                preferred_element_type=jnp.float32)

        gt = gates_ref[...]
        gi = jax.nn.sigmoid(gt[:, 0 * hdim:1 * hdim])
        gf = jax.nn.sigmoid(gt[:, 1 * hdim:2 * hdim])
        gg = jnp.tanh(gt[:, 2 * hdim:3 * hdim])
        go = jax.nn.sigmoid(gt[:, 3 * hdim:4 * hdim])
        c_new = gf * c + gi * gg
        h_new = go * jnp.tanh(c_new)
        live = kv_ref[G + g] > 0
        h = jnp.where(live, h_new, h)
        c = jnp.where(live, c_new, c)
    h_out[...] = h
    c_out[...] = c


def _make_lstm_chunk(NP, H):
    NB = NP // BN

    def xt_map(i, kv):
        last = jnp.maximum(pl.cdiv(kv[0], BN) - 1, 0)
        return (0, jnp.minimum(i, last), 0)

    grid_spec = pltpu.PrefetchScalarGridSpec(
        num_scalar_prefetch=1,
        grid=(NB,),
        in_specs=[
            pl.BlockSpec((G, BN, H), xt_map),
            pl.BlockSpec((BN, H), lambda i, kv: (i, 0)),
            pl.BlockSpec((BN, H), lambda i, kv: (i, 0)),
            pl.BlockSpec((H, 4 * H), lambda i, kv: (0, 0)),
            pl.BlockSpec((H, 4 * H), lambda i, kv: (0, 0)),
            pl.BlockSpec((1, 4 * H), lambda i, kv: (0, 0)),
        ],
        out_specs=[
            pl.BlockSpec((BN, H), lambda i, kv: (i, 0)),
            pl.BlockSpec((BN, H), lambda i, kv: (i, 0)),
        ],
        scratch_shapes=[pltpu.VMEM((BN, 4 * H), jnp.float32)],
    )
    return pl.pallas_call(
        _lstm_chunk_body,
        grid_spec=grid_spec,
        out_shape=[
            jax.ShapeDtypeStruct((NP, H), jnp.float32),
            jax.ShapeDtypeStruct((NP, H), jnp.float32),
        ],
        compiler_params=pltpu.CompilerParams(
            dimension_semantics=("arbitrary",)),
    )


def _tail_body(aggr_ref, h_ref, wl_ref, wr_ref, b_ref, o_ref, *, nvalid):
    i = pl.program_id(0)
    v = (jnp.dot(aggr_ref[...].astype(jnp.bfloat16), wl_ref[...],
                 preferred_element_type=jnp.float32)
         + jnp.dot(h_ref[...].astype(jnp.bfloat16), wr_ref[...],
                   preferred_element_type=jnp.float32)
         + b_ref[...])
    v = jnp.maximum(v, 0.0)
    rows = i * BT + lax.broadcasted_iota(jnp.int32, v.shape, 0)
    o_ref[...] = jnp.where(rows < nvalid, v, 0.0)


def _make_tail(NP, H, N):
    return pl.pallas_call(
        functools.partial(_tail_body, nvalid=N),
        grid=(NP // BT,),
        in_specs=[
            pl.BlockSpec((BT, H), lambda i: (i, 0)),
            pl.BlockSpec((BT, H), lambda i: (i, 0)),
            pl.BlockSpec((H, H), lambda i: (0, 0)),
            pl.BlockSpec((H, H), lambda i: (0, 0)),
            pl.BlockSpec((1, H), lambda i: (0, 0)),
        ],
        out_specs=pl.BlockSpec((BT, H), lambda i: (i, 0)),
        out_shape=jax.ShapeDtypeStruct((NP, H), jnp.float32),
        compiler_params=pltpu.CompilerParams(
            dimension_semantics=("arbitrary",)),
    )


def _mlp_body(hs_ref, hd_ref, ea_ref, w1s_ref, w1d_ref, w1e_ref, b1_ref,
              w2_ref, b2_ref, w3_ref, b3_ref, o_ref):
    z = (jnp.dot(hs_ref[...].astype(jnp.bfloat16), w1s_ref[...],
                 preferred_element_type=jnp.float32)
         + jnp.dot(hd_ref[...].astype(jnp.bfloat16), w1d_ref[...],
                   preferred_element_type=jnp.float32)
         + jnp.dot(ea_ref[...], w1e_ref[...],
                   preferred_element_type=jnp.float32)
         + b1_ref[...])
    z = jnp.maximum(z, 0.0).astype(jnp.bfloat16)
    z = jnp.maximum(
        jnp.dot(z, w2_ref[...], preferred_element_type=jnp.float32)
        + b2_ref[...], 0.0).astype(jnp.bfloat16)
    o_ref[...] = (jnp.dot(z, w3_ref[...], preferred_element_type=jnp.float32)
                  + b3_ref[...])


def _make_mlp(EP, H, ED, H2, OUT):
    return pl.pallas_call(
        _mlp_body,
        grid=(EP // BE,),
        in_specs=[
            pl.BlockSpec((BE, H), lambda i: (i, 0)),
            pl.BlockSpec((BE, H), lambda i: (EP // BE + i, 0)),
            pl.BlockSpec((BE, ED), lambda i: (i, 0)),
            pl.BlockSpec((H, H), lambda i: (0, 0)),
            pl.BlockSpec((H, H), lambda i: (0, 0)),
            pl.BlockSpec((ED, H), lambda i: (0, 0)),
            pl.BlockSpec((1, H), lambda i: (0, 0)),
            pl.BlockSpec((H, H2), lambda i: (0, 0)),
            pl.BlockSpec((1, H2), lambda i: (0, 0)),
            pl.BlockSpec((H2, OUT), lambda i: (0, 0)),
            pl.BlockSpec((1, OUT), lambda i: (0, 0)),
        ],
        out_specs=pl.BlockSpec((BE, OUT), lambda i: (i, 0)),
        out_shape=jax.ShapeDtypeStruct((EP, OUT), jnp.float32),
        compiler_params=pltpu.CompilerParams(
            dimension_semantics=("arbitrary",)),
    )


def kernel(x, edge_index, edge_attr, params):
    x = x.astype(jnp.float32)
    src = edge_index[0].astype(jnp.int32)
    dst = edge_index[1].astype(jnp.int32)
    N, D = x.shape
    E = src.shape[0]
    H = D
    NP = -(-N // 2560) * 2560
    EPAD = -(-E // (NW * CH_A * 2)) * (NW * CH_A * 2)
    EP = EPAD
    assert EP % BE == 0

    starts_all = jnp.searchsorted(
        dst, jnp.arange(N, dtype=jnp.int32)).astype(jnp.int32)
    counts = jnp.diff(jnp.concatenate(
        [starts_all, jnp.array([E], jnp.int32)]))
    T = counts.max().astype(jnp.int32)

    order = jnp.argsort(-counts).astype(jnp.int32)
    counts_s = jnp.concatenate(
        [counts[order], jnp.zeros((NP - N,), jnp.int32)])
    starts_s = jnp.concatenate(
        [starts_all[order], jnp.full((NP - N,), E - 1, jnp.int32)])
    counts_asc = counts_s[::-1]
    pos = jnp.argsort(order).astype(jnp.int32)
    pos_pad = jnp.concatenate([pos, jnp.full((NP - N,), N, jnp.int32)])
    x_s = jnp.concatenate([x[order], jnp.zeros((NP - N, D), jnp.float32)])
    x_unp = jnp.concatenate([x, jnp.zeros((NP - N, D), jnp.float32)])

    def pad_idx(ix):
        return jnp.concatenate(
            [ix, jnp.full((EPAD - E,), N, jnp.int32)]
        ).reshape(NW, EPAD // NW // CH_A, CH_A)

    idx_src = pad_idx(src)
    idx_dst = pad_idx(dst)

    gather_rows = _make_gather_rows(H, EPAD, jnp.float32)
    lstm_chunk = _make_lstm_chunk(NP, H)
    tail = _make_tail(NP, H, N)
    garange = jnp.arange(G, dtype=jnp.int32)

    def layer(h_in, h_unp, p):
        wihT = p['W_ih'].T.astype(jnp.bfloat16)
        whhT = p['W_hh'].T.astype(jnp.bfloat16)
        b = (p['b_ih'] + p['b_hh']).reshape(1, 4 * H)

        feats = gather_rows(h_unp, idx_src)

        def gather_chunk(t0):
            ts = t0 + garange
            gidx = jnp.minimum(starts_s[None, :] + ts[:, None], E - 1)
            valid = ts[:, None] < counts_s[None, :]
            return jnp.where(valid[..., None], feats[gidx], 0.0)

        def cond(carry):
            t0 = carry[0]
            return t0 < T

        def body(carry):
            t0, h, c, xt = carry
            xt_next = gather_chunk(t0 + G)
            ts = t0 + garange
            kvec = (NP - jnp.searchsorted(counts_asc, ts, side='right')
                    ).astype(jnp.int32)
            live = (ts < T).astype(jnp.int32)
            kv = jnp.concatenate([kvec, live])
            h, c = lstm_chunk(kv, xt, h, c, wihT, whhT, b)
            return t0 + G, h, c, xt_next

        z = jnp.zeros((NP, H), jnp.float32)
        _, hl, _, _ = lax.while_loop(
            cond, body, (jnp.int32(0), z, z, gather_chunk(jnp.int32(0))))
        return tail(hl, h_in, p['W_l'].T.astype(jnp.bfloat16),
                    p['W_r'].T.astype(jnp.bfloat16), p['b_l'].reshape(1, H))

    h = layer(x_s, x_unp, params['conv1'])
    h = layer(h, h[pos_pad], params['conv2'])
    h = layer(h, h[pos_pad], params['conv3'])
    h_unp = h[pos_pad]

    gather_rows2 = _make_gather_rows(H, 2 * EPAD, jnp.float32)
    idx_both = jnp.concatenate(
        [idx_src.reshape(-1), idx_dst.reshape(-1)]
    ).reshape(NW, 2 * EPAD // NW // CH_A, CH_A)
    hsd = gather_rows2(h_unp, idx_both)
    m = params['edge_mlp']
    H2 = m['W2'].shape[0]
    OUT = m['W3'].shape[0]
    ED = edge_attr.shape[1]
    w1 = m['W1'].T.astype(jnp.bfloat16)

    ea = jnp.concatenate(
        [edge_attr.astype(jnp.bfloat16),
         jnp.zeros((EP - E, ED), jnp.bfloat16)])

    mlp = _make_mlp(EP, H, ED, H2, OUT)
    out = mlp(hsd, hsd, ea, w1[:H], w1[H:2 * H], w1[2 * H:],
              m['b1'].reshape(1, H),
              m['W2'].T.astype(jnp.bfloat16), m['b2'].reshape(1, H2),
              m['W3'].T.astype(jnp.bfloat16), m['b3'].reshape(1, OUT))
    return out[:E]
